# baseline (speedup 1.0000x reference)
import functools

import jax
import jax.numpy as jnp
from jax import lax
from jax.experimental import pallas as pl
from jax.experimental.pallas import tpu as pltpu
from jax.experimental.pallas import tpu_sc as plsc

BATCH = 16384
EMBED_DIM = 32
NUM_CORES = 2
NUM_SUBCORES = 16
NUM_WORKERS = NUM_CORES * NUM_SUBCORES
BPW = BATCH // NUM_WORKERS
BLK = 2048


def _sc_gather_body(uidx, iidx, uemb, iemb, uout, iout,
                    uidx_v, iidx_v, urows_v, irows_v, usem, isem):
    wid = lax.axis_index("s") * NUM_CORES + lax.axis_index("c")
    base = wid * BPW
    pltpu.sync_copy(uidx.at[pl.ds(base, BPW)], uidx_v)
    pltpu.sync_copy(iidx.at[pl.ds(base, BPW)], iidx_v)
    ucp = pltpu.async_copy(uemb.at[uidx_v], urows_v, usem)
    icp = pltpu.async_copy(iemb.at[iidx_v], irows_v, isem)
    ucp.wait()
    pltpu.sync_copy(urows_v, uout.at[pl.ds(base, BPW)])
    icp.wait()
    pltpu.sync_copy(irows_v, iout.at[pl.ds(base, BPW)])


def _sc_gather(user_idx, item_idx, user_emb, item_emb):
    mesh = plsc.VectorSubcoreMesh(
        core_axis_name="c", subcore_axis_name="s",
        num_cores=NUM_CORES, num_subcores=NUM_SUBCORES)
    f = pl.kernel(
        _sc_gather_body,
        out_type=[
            jax.ShapeDtypeStruct((BATCH, EMBED_DIM), jnp.float32),
            jax.ShapeDtypeStruct((BATCH, EMBED_DIM), jnp.float32),
        ],
        mesh=mesh,
        scratch_types=[
            pltpu.VMEM((BPW,), jnp.int32),
            pltpu.VMEM((BPW,), jnp.int32),
            pltpu.VMEM((BPW, EMBED_DIM), jnp.float32),
            pltpu.VMEM((BPW, EMBED_DIM), jnp.float32),
            pltpu.SemaphoreType.DMA,
            pltpu.SemaphoreType.DMA,
        ],
        compiler_params=pltpu.CompilerParams(use_tc_tiling_on_sc=False),
    )
    return f(user_idx, item_idx, user_emb, item_emb)


def _mlp_body(uv_ref, iv_ref, aid_ref, gid_ref, aemb_ref, gemb_ref,
              w1_ref, b1_ref, w2_ref, b2_ref, w3_ref, b3_ref,
              wo_ref, bo_ref, out_ref):
    f32 = jnp.float32

    def dgt(x, w):
        return lax.dot_general(x, w, (((1,), (1,)), ((), ())),
                               preferred_element_type=f32)

    aid = aid_ref[...]
    gid = gid_ref[...]
    a_oh = (aid == lax.broadcasted_iota(jnp.int32, (1, 10), 1)).astype(f32)
    g_oh = (gid == lax.broadcasted_iota(jnp.int32, (1, 2), 1)).astype(f32)
    av = jnp.dot(a_oh, aemb_ref[...], preferred_element_type=f32)
    gv = jnp.dot(g_oh, gemb_ref[...], preferred_element_type=f32)
    w1 = w1_ref[...]
    h = (dgt(uv_ref[...], w1[:, 0:32]) + dgt(iv_ref[...], w1[:, 32:64])
         + dgt(av, w1[:, 64:96]) + dgt(gv, w1[:, 96:128]) + b1_ref[...])
    x = jnp.maximum(h, 0.0)
    x = jnp.maximum(dgt(x, w2_ref[...]) + b2_ref[...], 0.0)
    x = jnp.maximum(dgt(x, w3_ref[...]) + b3_ref[...], 0.0)
    o = jnp.sum(x * wo_ref[...], axis=1, keepdims=True) + bo_ref[0, 0]
    out_ref[...] = 1.0 / (1.0 + jnp.exp(-o))


def _mlp(uv, iv, aid, gid, age_emb, gender_emb, W1, b1, W2, b2, W3, b3,
         Wo, bo, interpret=False):
    nblk = BATCH // BLK
    full = lambda shape: pl.BlockSpec(shape, lambda i: (0, 0))
    batch_blk = lambda w: pl.BlockSpec((BLK, w), lambda i: (i, 0))
    return pl.pallas_call(
        _mlp_body,
        grid=(nblk,),
        in_specs=[
            batch_blk(EMBED_DIM),
            batch_blk(EMBED_DIM),
            batch_blk(1),
            batch_blk(1),
            full((10, EMBED_DIM)),
            full((2, EMBED_DIM)),
            full((64, 128)),
            full((1, 64)),
            full((32, 64)),
            full((1, 32)),
            full((16, 32)),
            full((1, 16)),
            full((1, 16)),
            pl.BlockSpec(memory_space=pltpu.SMEM),
        ],
        out_specs=batch_blk(1),
        out_shape=jax.ShapeDtypeStruct((BATCH, 1), jnp.float32),
        interpret=interpret,
    )(uv, iv, aid, gid, age_emb, gender_emb, W1, b1, W2, b2, W3, b3, Wo, bo)


@jax.jit
def kernel(user_input, item_input, age_input, gender_input, user_emb,
           item_emb, age_emb, gender_emb, W1, b1, W2, b2, W3, b3, Wo, bo):
    uidx = user_input.astype(jnp.int32)
    iidx = item_input.astype(jnp.int32)
    uv, iv = _sc_gather(uidx, iidx, user_emb, item_emb)
    aid = age_input.astype(jnp.int32).reshape(BATCH, 1)
    gid = gender_input.astype(jnp.int32).reshape(BATCH, 1)
    return _mlp(uv, iv, aid, gid, age_emb, gender_emb,
                W1, b1.reshape(1, 64), W2, b2.reshape(1, 32),
                W3, b3.reshape(1, 16), Wo, bo.reshape(1, 1))
